# Initial kernel scaffold; baseline (speedup 1.0000x reference)
import jax
import jax.numpy as jnp
from jax.experimental import pallas as pl


def kernel(params, x, edge_index, bi_edge_index, y, batch, hist_mask, pos_mask, neg_masks, hist_seqs, hist_seq_lens, pos_seq, pos_seq_len, neg_seqs, neg_seq_lens, title_table):
    raise NotImplementedError("write your pallas kernel here")


# speedup vs baseline: 14.3441x; 14.3441x over previous
import functools

import jax
import jax.numpy as jnp
from jax import lax
from jax.experimental import pallas as pl
from jax.experimental.pallas import tpu as pltpu
from jax.experimental.pallas import tpu_sc as plsc

N = 10000
EG = 320000
B = 64
H = 50
NEG = 4
L = 20
D = 128
NNEWS = 100000
NEWS_N = B * (H + NEG + 1)
EBI = 32768

_NW = 32
_NT = 16


@functools.lru_cache(maxsize=None)
def _mesh():
    return plsc.VectorSubcoreMesh(core_axis_name="c", subcore_axis_name="s")


@functools.lru_cache(maxsize=None)
def _make_sc_gather(V, Dr, Mp, dtype_name):
    dtype = jnp.dtype(dtype_name)
    BW = Mp // _NW
    ncg = BW // 128
    assert BW % 128 == 0

    @functools.partial(
        pl.kernel,
        out_type=jax.ShapeDtypeStruct((Mp, Dr), dtype),
        mesh=_mesh(),
        scratch_types=[
            pltpu.VMEM((ncg, 128), jnp.int32),
            pltpu.VMEM((128, Dr), dtype),
            pltpu.SemaphoreType.DMA,
        ],
        compiler_params=pltpu.CompilerParams(use_tc_tiling_on_sc=False),
    )
    def k(table, idx, out, idx_v, rows, sem):
        c = lax.axis_index("c")
        s = lax.axis_index("s")
        wid = c * _NT + s
        pltpu.sync_copy(idx.at[wid], idx_v)

        def body(i, _):
            pltpu.async_copy(table.at[idx_v.at[i]], rows, sem).wait()
            pltpu.sync_copy(rows, out.at[pl.ds((wid * ncg + i) * 128, 128)])
            return 0

        lax.fori_loop(0, ncg, body, 0)

    return k


def _sc_gather(table, idx):
    V, Dr = table.shape
    M = idx.shape[0]
    Mp = ((M + _NW * 128 - 1) // (_NW * 128)) * (_NW * 128)
    idx_p = jnp.zeros((Mp,), jnp.int32).at[:M].set(idx.astype(jnp.int32))
    idx_p = idx_p.reshape(_NW, Mp // (_NW * 128), 128)
    out = _make_sc_gather(V, Dr, Mp, jnp.dtype(table.dtype).name)(table, idx_p)
    return out[:M]


_SE = N
_EWM = EG // _NW
_KE = 80
_NCM = _EWM // _KE
_NDP = ((N + 127) // 128) * 128
_RPT = _NDP // _NT


_GE = 16


@functools.lru_cache(maxsize=None)
def _make_sc_edge():
    K = _KE
    nfull, nrem = _RPT // K, _RPT % K
    GK = _GE * K

    @functools.partial(
        pl.kernel,
        out_type=(
            jax.ShapeDtypeStruct((2, _NDP, D), jnp.float32),
            jax.ShapeDtypeStruct((2, _NDP), jnp.float32),
        ),
        mesh=_mesh(),
        scratch_types=[
            pltpu.VMEM((_SE,), jnp.float32),
            pltpu.VMEM((_SE,), jnp.float32),
            pltpu.VMEM((GK,), jnp.int32),
            pltpu.VMEM((_GE, K), jnp.int32),
            pltpu.VMEM((K,), jnp.float32),
            pltpu.VMEM((K, D), jnp.float32),
            pltpu.VMEM((128,), jnp.float32),
            pltpu.VMEM((16,), jnp.int32),
            pltpu.VMEM_SHARED((_NDP, D), jnp.float32),
            pltpu.VMEM_SHARED((_NDP,), jnp.float32),
            pltpu.SemaphoreType.DMA,
        ],
        compiler_params=pltpu.CompilerParams(
            use_tc_tiling_on_sc=False, needs_layout_passes=False),
    )
    def k(h_hbm, asv, adv, src_h, dst_h, cnt_h, p_out, s_out,
          asv_l, adv_l, src_g, dst_g, exk, rows, zv128, cnt, acc, s_sh, sem):
        c = lax.axis_index("c")
        sid = lax.axis_index("s")
        wid = c * _NT + sid

        pltpu.sync_copy(asv, asv_l)
        pltpu.sync_copy(adv, adv_l)
        pltpu.sync_copy(cnt_h, cnt)

        zv = jnp.zeros((16,), jnp.float32)

        def zrow(i, _):
            for r in range(D // 16):
                rows[i, pl.ds(r * 16, 16)] = zv
            return 0

        lax.fori_loop(0, K, zrow, 0)
        for r in range(8):
            zv128[pl.ds(r * 16, 16)] = zv

        base = sid * _RPT
        for j in range(nfull):
            pltpu.sync_copy(rows, acc.at[pl.ds(base + j * K, K)])
        if nrem:
            pltpu.sync_copy(rows.at[pl.ds(0, nrem)],
                            acc.at[pl.ds(base + nfull * K, nrem)])
        for j in range(_RPT // 128):
            pltpu.sync_copy(zv128, s_sh.at[pl.ds(base + j * 128, 128)])
        srem = _RPT % 128
        if srem:
            pltpu.sync_copy(zv128.at[pl.ds(0, srem)],
                            s_sh.at[pl.ds(base + (_RPT // 128) * 128, srem)])

        plsc.subcore_barrier()

        ew = jnp.max(cnt[...])
        ncd = (ew + (K - 1)) // K
        ngrp = (ncd + (_GE - 1)) // _GE
        lane = lax.iota(jnp.int32, 16)

        def group(g, _):
            pltpu.sync_copy(
                src_h.at[pl.ds(wid * _EWM + g * GK, GK)], src_g)
            pltpu.sync_copy(dst_h.at[wid, pl.ds(g * _GE, _GE)], dst_g)
            nloc = jnp.minimum(ncd - g * _GE, _GE)

            def chunk(cl, _):
                for t in range(K // 16):
                    pos = cl * K + t * 16
                    s16 = src_g[pl.ds(pos, 16)]
                    d16 = dst_g[cl, pl.ds(t * 16, 16)]
                    a16 = plsc.load_gather(asv_l, [s16])
                    b16 = plsc.load_gather(adv_l, [d16])
                    lg = a16 + b16
                    lg = jnp.where(lg > 0, lg, 0.2 * lg)
                    gpos = (g * _GE + cl) * K + t * 16
                    e16 = jnp.where(gpos + lane < ew, jnp.exp(lg), 0.0)
                    exk[pl.ds(t * 16, 16)] = e16
                pltpu.async_copy(
                    h_hbm.at[src_g.at[pl.ds(cl * K, K)]], rows, sem).wait()

                def scale(i, _):
                    w16 = plsc.load_gather(
                        exk, [jnp.full((16,), i, jnp.int32)])
                    for r in range(D // 16):
                        sl = pl.ds(r * 16, 16)
                        rows[i, sl] = rows[i, sl] * w16
                    return 0

                lax.fori_loop(0, K, scale, 0)
                pltpu.sync_copy(rows, acc.at[dst_g.at[cl]], add=True)
                pltpu.sync_copy(exk, s_sh.at[dst_g.at[cl]], add=True)
                return 0

            lax.fori_loop(0, nloc, chunk, 0)
            return 0

        lax.fori_loop(0, ngrp, group, 0)
        plsc.subcore_barrier()

        pltpu.sync_copy(acc.at[pl.ds(base, _RPT)],
                        p_out.at[c, pl.ds(base, _RPT)])

        @pl.when(sid == 0)
        def _():
            pltpu.sync_copy(s_sh, s_out.at[c])

    return k


def _sc_edge(h, asv, adv, src, dst, Nd, E, K_unused=None):
    S = h.shape[0]
    if S < _SE:
        h = jnp.pad(h, ((0, _SE - S), (0, 0)))
        asv = jnp.pad(asv, (0, _SE - S))
    if Nd < _SE:
        adv = jnp.pad(adv, (0, _SE - Nd))
    ew = E // _NW
    if ew < _EWM:
        src = jnp.pad(src.reshape(_NW, ew), ((0, 0), (0, _EWM - ew)))
        dst = jnp.pad(dst.reshape(_NW, ew), ((0, 0), (0, _EWM - ew)))
    src = src.reshape(_NW * _EWM)
    dst_r = dst.reshape(_NW, _NCM, _KE)
    cnt = jnp.full((16,), ew, jnp.int32)
    P, s2 = _make_sc_edge()(h, asv, adv, src, dst_r, cnt)
    return P[:, :Nd], s2[:, :Nd]


def _lin(x, W, b, act):
    M = x.shape[0]

    def kern(x_ref, w_ref, b_ref, o_ref):
        y = jnp.dot(x_ref[...], w_ref[...], preferred_element_type=jnp.float32)
        if b_ref is not None:
            y = y + b_ref[...]
        if act:
            y = jnp.tanh(y)
        o_ref[...] = y

    if b is None:
        f = lambda x_ref, w_ref, o_ref: kern(x_ref, w_ref, None, o_ref)
        return pl.pallas_call(
            f, out_shape=jax.ShapeDtypeStruct((M, D), jnp.float32))(x, W)
    return pl.pallas_call(
        kern, out_shape=jax.ShapeDtypeStruct((M, D), jnp.float32))(
            x, W, b.reshape(1, D))


def _lin2(x, W, a8):
    M = x.shape[0]

    def kern(x_ref, w_ref, a_ref, h_ref, av_ref):
        h = jnp.dot(x_ref[...], w_ref[...], preferred_element_type=jnp.float32)
        h_ref[...] = h
        av_ref[...] = jnp.dot(h, a_ref[...], preferred_element_type=jnp.float32)

    return pl.pallas_call(
        kern,
        out_shape=(jax.ShapeDtypeStruct((M, D), jnp.float32),
                   jax.ShapeDtypeStruct((M, 8), jnp.float32)))(x, W, a8)


def _combine(P, s2):
    Nd = P.shape[1]

    def kern(p_ref, s_ref, o_ref):
        s = s_ref[0, :] + s_ref[1, :]
        o_ref[...] = (p_ref[0] + p_ref[1]) / (s[:, None] + 1e-16)

    return pl.pallas_call(
        kern, out_shape=jax.ShapeDtypeStruct((Nd, D), jnp.float32))(P, s2)


def _wpool(emb, lens):
    M = emb.shape[0]
    BM = 320
    assert M % BM == 0

    def kern(e_ref, l_ref, o_ref):
        ln = l_ref[...]
        m = (lax.broadcasted_iota(jnp.int32, (BM, L), 1).astype(jnp.float32)
             < ln).astype(jnp.float32)
        o_ref[...] = jnp.sum(e_ref[...] * m[..., None], axis=1) / jnp.maximum(
            ln, 1.0)

    return pl.pallas_call(
        kern,
        grid=(M // BM,),
        in_specs=[
            pl.BlockSpec((BM, L, D), lambda i: (i, 0, 0)),
            pl.BlockSpec((BM, 1), lambda i: (i, 0)),
        ],
        out_specs=pl.BlockSpec((BM, D), lambda i: (i, 0)),
        out_shape=jax.ShapeDtypeStruct((M, D), jnp.float32),
    )(emb, lens)


def _bpool(node, wg, bg, batch, masks):
    def kern(n_ref, wg_ref, bg_ref, b_ref, m_ref, o_ref):
        node_v = n_ref[...]
        gate = jnp.sum(node_v * wg_ref[...], axis=1, keepdims=True) + bg_ref[0, 0]
        oh = (b_ref[...] == lax.broadcasted_iota(jnp.int32, (N, B), 1))
        ohf = oh.astype(jnp.float32)
        for i in range(NEG + 2):
            mask_c = m_ref[:, i][:, None]
            lg = jnp.where(mask_c > 0, gate, -1e9)
            mseg = jnp.max(jnp.where(oh, lg, -3.4e38), axis=0, keepdims=True)
            mseg = jnp.where(mseg > -1e38, mseg, 0.0)
            mn = jnp.sum(ohf * mseg, axis=1, keepdims=True)
            ex = jnp.exp(lg - mn)
            sb = jax.lax.dot_general(ex, ohf, (((0,), (0,)), ((), ())),
                                     preferred_element_type=jnp.float32)
            sn = jnp.sum(ohf * sb, axis=1, keepdims=True)
            al = ex / (sn + 1e-16)
            w8 = ohf * al
            col = jax.lax.dot_general(w8, node_v, (((0,), (0,)), ((), ())),
                                      preferred_element_type=jnp.float32)
            o_ref[:, i, :] = col

    return pl.pallas_call(
        kern, out_shape=jax.ShapeDtypeStruct((B, NEG + 2, D), jnp.float32))(
            node, wg, bg, batch, masks)


def _head(uth3, Wsa, bsa, vsa, pooled, tth, wa, wb, bwt):
    def kern(u_ref, wsa_ref, bsa_ref, vsa_ref, p_ref, t_ref, wa_ref, wb_ref,
             bwt_ref, o_ref):
        u = u_ref[...]
        t1 = jnp.tanh(
            jnp.dot(u, wsa_ref[...], preferred_element_type=jnp.float32)
            + bsa_ref[...])
        sc = jnp.sum(t1 * vsa_ref[...], axis=1).reshape(B, H)
        m = jnp.max(sc, axis=1, keepdims=True)
        ex = jnp.exp(sc - m)
        aw = ex / jnp.sum(ex, axis=1, keepdims=True)
        u3 = u.reshape(B, H, D)
        u_title = jnp.sum(u3 * aw[:, :, None], axis=1)
        u_graph = p_ref[:, 0, :]
        t_graph = p_ref[:, 1:, :].reshape(B * (NEG + 1), D)
        u_g5 = jnp.broadcast_to(u_graph[:, None, :],
                                (B, NEG + 1, D)).reshape(B * (NEG + 1), D)
        u_t5 = jnp.broadcast_to(u_title[:, None, :],
                                (B, NEG + 1, D)).reshape(B * (NEG + 1), D)
        bwt = bwt_ref[0, 0]
        uw = jax.nn.sigmoid(
            jnp.sum(u_g5 * wa_ref[...], axis=1, keepdims=True)
            + jnp.sum(u_t5 * wb_ref[...], axis=1, keepdims=True) + bwt)
        u_hid = uw * u_g5 + (1.0 - uw) * u_t5
        t_f = t_ref[...]
        tw = jax.nn.sigmoid(
            jnp.sum(t_graph * wa_ref[...], axis=1, keepdims=True)
            + jnp.sum(t_f * wb_ref[...], axis=1, keepdims=True) + bwt)
        t_hid = tw * t_graph + (1.0 - tw) * t_f
        o_ref[...] = jnp.sum(u_hid * t_hid, axis=1).reshape(B, NEG + 1)

    return pl.pallas_call(
        kern, out_shape=jax.ShapeDtypeStruct((B, NEG + 1), jnp.float32))(
            uth3, Wsa, bsa, vsa, pooled, tth, wa, wb, bwt)


def _a8(u, v):
    return jnp.concatenate(
        [u[:, None], v[:, None], jnp.zeros((D, 6), jnp.float32)], axis=1)


def _cross_pair(xs, xd, Ws, Wd, cs, cd, src, dst, Nd, E, K):
    hs, avs = _lin2(xs, Ws, _a8(cs, cs))
    hd, avd = _lin2(xd, Wd, _a8(cd, cd))
    P, s2 = _sc_edge(hs, avs[:, 0], avd[:, 0], src, dst, Nd, E, K)
    return _combine(P, s2)


def kernel(params, x, edge_index, bi_edge_index, y, batch, hist_mask,
           pos_mask, neg_masks, hist_seqs, hist_seq_lens, pos_seq,
           pos_seq_len, neg_seqs, neg_seq_lens, title_table):
    p = params
    src = edge_index[0].astype(jnp.int32)
    dst = edge_index[1].astype(jnp.int32)
    bi_src = bi_edge_index[0].astype(jnp.int32)
    bi_dst = bi_edge_index[1].astype(jnp.int32)

    all_seqs = jnp.concatenate(
        [hist_seqs.reshape(-1), pos_seq[:, 0], neg_seqs.reshape(-1)])
    all_lens = jnp.concatenate(
        [hist_seq_lens.reshape(-1), pos_seq_len, neg_seq_lens.reshape(-1)])
    tt_pad = jnp.pad(title_table.astype(jnp.int32), ((0, 0), (0, 12)))
    ids3 = _sc_gather(tt_pad, all_seqs)[:, :L]
    emb = _sc_gather(p['wt'], ids3.reshape(-1))
    pooled_t = _wpool(emb.reshape(NEWS_N, L, D),
                      all_lens.astype(jnp.float32).reshape(NEWS_N, 1))
    titles = _lin(pooled_t, p['Wt'], p['bt'], True)
    uth = _lin(titles[:B * H], p['Wu1'], p['bu1'], True)
    tth = jnp.concatenate(
        [titles[B * H:B * H + B].reshape(B, 1, D),
         titles[B * H + B:].reshape(B, NEG, D)], axis=1)
    news_h = jnp.concatenate(
        [uth.reshape(B, H, D), tth], axis=1).reshape(NEWS_N, D)

    node0 = _sc_gather(p['ent'], x[:, 0].astype(jnp.int32))

    h1, av1 = _lin2(node0, p['Wg1'], _a8(p['as1'], p['ad1']))
    P1, s1 = _sc_edge(h1, av1[:, 0], av1[:, 1], src, dst, N, EG, 80)
    node_h = _combine(P1, s1)

    node_c = _cross_pair(news_h, node_h, p['Ws1'], p['Wd1'], p['cs1'],
                         p['cd1'], bi_src, bi_dst, N, EBI, 64)
    news_c = _cross_pair(node_h, news_h, p['Ws1'], p['Wd1'], p['cs1'],
                         p['cd1'], bi_dst, bi_src, NEWS_N, EBI, 64)

    nc3 = news_c.reshape(B, H + NEG + 1, D)
    uth2 = _lin(nc3[:, :H].reshape(B * H, D), p['Wu2'], p['bu2'], True)
    news_h2 = jnp.concatenate(
        [uth2.reshape(B, H, D), nc3[:, H:]], axis=1).reshape(NEWS_N, D)

    h2, av2 = _lin2(node_c, p['Wg2'], _a8(p['as2'], p['ad2']))
    P2, s2 = _sc_edge(h2, av2[:, 0], av2[:, 1], src, dst, N, EG, 80)
    node_h2 = _combine(P2, s2)

    node_c2 = _cross_pair(news_h2, node_h2, p['Ws2'], p['Wd2'], p['cs2'],
                          p['cd2'], bi_src, bi_dst, N, EBI, 64)
    news_c2 = _cross_pair(node_h2, news_h2, p['Ws2'], p['Wd2'], p['cs2'],
                          p['cd2'], bi_dst, bi_src, NEWS_N, EBI, 64)

    nc23 = news_c2.reshape(B, H + NEG + 1, D)
    uth3 = _lin(nc23[:, :H].reshape(B * H, D), p['Wu3'], p['bu3'], True)
    tth3 = nc23[:, H:].reshape(B * (NEG + 1), D)

    masks6 = jnp.concatenate(
        [hist_mask, pos_mask, neg_masks], axis=1).astype(jnp.int32)
    pooled = _bpool(node_c2, p['Wgate'][:, 0].reshape(1, D),
                    p['bgate'].reshape(1, 1),
                    batch.astype(jnp.int32).reshape(N, 1), masks6)
    logits = _head(uth3, p['Wsa'], p['bsa'].reshape(1, D),
                   p['vsa'].reshape(1, D), pooled, tth3,
                   p['Wwt'][:D, 0].reshape(1, D), p['Wwt'][D:, 0].reshape(1, D),
                   p['bwt'].reshape(1, 1))
    return logits


# speedup vs baseline: 17.3666x; 1.2107x over previous
import functools

import jax
import jax.numpy as jnp
from jax import lax
from jax.experimental import pallas as pl
from jax.experimental.pallas import tpu as pltpu
from jax.experimental.pallas import tpu_sc as plsc

N = 10000
EG = 320000
B = 64
H = 50
NEG = 4
L = 20
D = 128
NNEWS = 100000
NEWS_N = B * (H + NEG + 1)
EBI = 32768

_NW = 32
_NT = 16


@functools.lru_cache(maxsize=None)
def _mesh():
    return plsc.VectorSubcoreMesh(core_axis_name="c", subcore_axis_name="s")


@functools.lru_cache(maxsize=None)
def _make_sc_gather(V, Dr, Mp, dtype_name):
    dtype = jnp.dtype(dtype_name)
    BW = Mp // _NW
    ncg = BW // 128
    assert BW % 128 == 0

    @functools.partial(
        pl.kernel,
        out_type=jax.ShapeDtypeStruct((Mp, Dr), dtype),
        mesh=_mesh(),
        scratch_types=[
            pltpu.VMEM((ncg, 128), jnp.int32),
            pltpu.VMEM((128, Dr), dtype),
            pltpu.VMEM((128, Dr), dtype),
            pltpu.SemaphoreType.DMA,
            pltpu.SemaphoreType.DMA,
        ],
        compiler_params=pltpu.CompilerParams(use_tc_tiling_on_sc=False),
    )
    def k(table, idx, out, idx_v, rows_a, rows_b, sem_a, sem_b):
        c = lax.axis_index("c")
        s = lax.axis_index("s")
        wid = c * _NT + s
        pltpu.sync_copy(idx.at[wid], idx_v)

        bufs = [(rows_a, sem_a), (rows_b, sem_b)]
        pltpu.async_copy(table.at[idx_v.at[0]], rows_a, sem_a)
        for i in range(ncg):
            cur, csem = bufs[i % 2]
            if i + 1 < ncg:
                nxt, nsem = bufs[(i + 1) % 2]
                pltpu.async_copy(table.at[idx_v.at[i + 1]], nxt, nsem)
            pltpu.make_async_copy(table.at[idx_v.at[i]], cur, csem).wait()
            pltpu.sync_copy(cur, out.at[pl.ds((wid * ncg + i) * 128, 128)])

    return k


def _sc_gather(table, idx):
    V, Dr = table.shape
    M = idx.shape[0]
    Mp = ((M + _NW * 128 - 1) // (_NW * 128)) * (_NW * 128)
    idx_p = jnp.zeros((Mp,), jnp.int32).at[:M].set(idx.astype(jnp.int32))
    idx_p = idx_p.reshape(_NW, Mp // (_NW * 128), 128)
    out = _make_sc_gather(V, Dr, Mp, jnp.dtype(table.dtype).name)(table, idx_p)
    return out[:M]


_SE = N
_EWM = EG // _NW
_KE = 80
_NCM = _EWM // _KE
_NDP = ((N + 127) // 128) * 128
_RPT = _NDP // _NT


_GE = 16


@functools.lru_cache(maxsize=None)
def _make_sc_edge():
    K = _KE
    nfull, nrem = _RPT // K, _RPT % K
    GK = _GE * K

    @functools.partial(
        pl.kernel,
        out_type=(
            jax.ShapeDtypeStruct((2, _NDP, D), jnp.float32),
            jax.ShapeDtypeStruct((2, _NDP), jnp.float32),
        ),
        mesh=_mesh(),
        scratch_types=[
            pltpu.VMEM((_SE,), jnp.float32),
            pltpu.VMEM((_SE,), jnp.float32),
            pltpu.VMEM((GK,), jnp.int32),
            pltpu.VMEM((_GE, K), jnp.int32),
            pltpu.VMEM((K,), jnp.float32),
            pltpu.VMEM((K, D), jnp.float32),
            pltpu.VMEM((K, D), jnp.float32),
            pltpu.VMEM((128,), jnp.float32),
            pltpu.VMEM((16,), jnp.int32),
            pltpu.VMEM_SHARED((_NDP, D), jnp.float32),
            pltpu.VMEM_SHARED((_NDP,), jnp.float32),
            pltpu.SemaphoreType.DMA,
            pltpu.SemaphoreType.DMA,
        ],
        compiler_params=pltpu.CompilerParams(
            use_tc_tiling_on_sc=False, needs_layout_passes=False),
    )
    def k(h_hbm, asv, adv, src_h, dst_h, cnt_h, p_out, s_out,
          asv_l, adv_l, src_g, dst_g, exk, rows, rows_b, zv128, cnt, acc,
          s_sh, sem, sem_b):
        c = lax.axis_index("c")
        sid = lax.axis_index("s")
        wid = c * _NT + sid

        pltpu.sync_copy(asv, asv_l)
        pltpu.sync_copy(adv, adv_l)
        pltpu.sync_copy(cnt_h, cnt)

        zv = jnp.zeros((16,), jnp.float32)

        def zrow(i, _):
            for r in range(D // 16):
                rows[i, pl.ds(r * 16, 16)] = zv
            return 0

        lax.fori_loop(0, K, zrow, 0)
        for r in range(8):
            zv128[pl.ds(r * 16, 16)] = zv

        base = sid * _RPT
        for j in range(nfull):
            pltpu.sync_copy(rows, acc.at[pl.ds(base + j * K, K)])
        if nrem:
            pltpu.sync_copy(rows.at[pl.ds(0, nrem)],
                            acc.at[pl.ds(base + nfull * K, nrem)])
        for j in range(_RPT // 128):
            pltpu.sync_copy(zv128, s_sh.at[pl.ds(base + j * 128, 128)])
        srem = _RPT % 128
        if srem:
            pltpu.sync_copy(zv128.at[pl.ds(0, srem)],
                            s_sh.at[pl.ds(base + (_RPT // 128) * 128, srem)])

        plsc.subcore_barrier()

        ew = jnp.max(cnt[...])
        ncd = (ew + (K - 1)) // K
        ngrp = (ncd + (_GE - 1)) // _GE
        lane = lax.iota(jnp.int32, 16)

        bufs = [(rows, sem), (rows_b, sem_b)]

        def group(g, _):
            pltpu.sync_copy(
                src_h.at[pl.ds(wid * _EWM + g * GK, GK)], src_g)
            pltpu.sync_copy(dst_h.at[wid, pl.ds(g * _GE, _GE)], dst_g)
            nloc = ncd - g * _GE
            pltpu.async_copy(h_hbm.at[src_g.at[pl.ds(0, K)]], rows, sem)

            for cl in range(_GE):
                cur, csem = bufs[cl % 2]
                nxt, nsem = bufs[(cl + 1) % 2]

                @pl.when(cl + 1 < nloc)
                def _(nxt=nxt, nsem=nsem, cl=cl):
                    if cl + 1 < _GE:
                        pltpu.async_copy(
                            h_hbm.at[src_g.at[pl.ds((cl + 1) * K, K)]],
                            nxt, nsem)

                @pl.when(cl < nloc)
                def _(cur=cur, csem=csem, cl=cl):
                    for t in range(K // 16):
                        pos = cl * K + t * 16
                        s16 = src_g[pl.ds(pos, 16)]
                        d16 = dst_g[cl, pl.ds(t * 16, 16)]
                        a16 = plsc.load_gather(asv_l, [s16])
                        b16 = plsc.load_gather(adv_l, [d16])
                        lg = a16 + b16
                        lg = jnp.where(lg > 0, lg, 0.2 * lg)
                        gpos = (g * _GE + cl) * K + t * 16
                        e16 = jnp.where(gpos + lane < ew, jnp.exp(lg), 0.0)
                        exk[pl.ds(t * 16, 16)] = e16
                    pltpu.make_async_copy(
                        h_hbm.at[src_g.at[pl.ds(cl * K, K)]],
                        cur, csem).wait()

                    def scale(i, _):
                        w16 = plsc.load_gather(
                            exk, [jnp.full((16,), i, jnp.int32)])
                        for r in range(D // 16):
                            sl = pl.ds(r * 16, 16)
                            cur[i, sl] = cur[i, sl] * w16
                        return 0

                    lax.fori_loop(0, K, scale, 0)
                    pltpu.sync_copy(cur, acc.at[dst_g.at[cl]], add=True)
                    pltpu.sync_copy(exk, s_sh.at[dst_g.at[cl]], add=True)
            return 0

        lax.fori_loop(0, ngrp, group, 0)
        plsc.subcore_barrier()

        pltpu.sync_copy(acc.at[pl.ds(base, _RPT)],
                        p_out.at[c, pl.ds(base, _RPT)])

        @pl.when(sid == 0)
        def _():
            pltpu.sync_copy(s_sh, s_out.at[c])

    return k


def _sc_edge(h, asv, adv, src, dst, Nd, E, K_unused=None):
    S = h.shape[0]
    if S < _SE:
        h = jnp.pad(h, ((0, _SE - S), (0, 0)))
        asv = jnp.pad(asv, (0, _SE - S))
    if Nd < _SE:
        adv = jnp.pad(adv, (0, _SE - Nd))
    ew = E // _NW
    if ew < _EWM:
        src = jnp.pad(src.reshape(_NW, ew), ((0, 0), (0, _EWM - ew)))
        dst = jnp.pad(dst.reshape(_NW, ew), ((0, 0), (0, _EWM - ew)))
    src = src.reshape(_NW * _EWM)
    dst_r = dst.reshape(_NW, _NCM, _KE)
    cnt = jnp.full((16,), ew, jnp.int32)
    P, s2 = _make_sc_edge()(h, asv, adv, src, dst_r, cnt)
    return P[:, :Nd], s2[:, :Nd]


def _lin(x, W, b, act):
    M = x.shape[0]

    def kern(x_ref, w_ref, b_ref, o_ref):
        y = jnp.dot(x_ref[...], w_ref[...], preferred_element_type=jnp.float32)
        if b_ref is not None:
            y = y + b_ref[...]
        if act:
            y = jnp.tanh(y)
        o_ref[...] = y

    if b is None:
        f = lambda x_ref, w_ref, o_ref: kern(x_ref, w_ref, None, o_ref)
        return pl.pallas_call(
            f, out_shape=jax.ShapeDtypeStruct((M, D), jnp.float32))(x, W)
    return pl.pallas_call(
        kern, out_shape=jax.ShapeDtypeStruct((M, D), jnp.float32))(
            x, W, b.reshape(1, D))


def _lin2(x, W, a8):
    M = x.shape[0]

    def kern(x_ref, w_ref, a_ref, h_ref, av_ref):
        h = jnp.dot(x_ref[...], w_ref[...], preferred_element_type=jnp.float32)
        h_ref[...] = h
        av_ref[...] = jnp.dot(h, a_ref[...], preferred_element_type=jnp.float32)

    return pl.pallas_call(
        kern,
        out_shape=(jax.ShapeDtypeStruct((M, D), jnp.float32),
                   jax.ShapeDtypeStruct((M, 8), jnp.float32)))(x, W, a8)


def _combine(P, s2):
    Nd = P.shape[1]

    def kern(p_ref, s_ref, o_ref):
        s = s_ref[0, :] + s_ref[1, :]
        o_ref[...] = (p_ref[0] + p_ref[1]) / (s[:, None] + 1e-16)

    return pl.pallas_call(
        kern, out_shape=jax.ShapeDtypeStruct((Nd, D), jnp.float32))(P, s2)


def _wpool(emb, lens):
    M = emb.shape[0]
    BM = 320
    assert M % BM == 0

    def kern(e_ref, l_ref, o_ref):
        ln = l_ref[...]
        m = (lax.broadcasted_iota(jnp.int32, (BM, L), 1).astype(jnp.float32)
             < ln).astype(jnp.float32)
        o_ref[...] = jnp.sum(e_ref[...] * m[..., None], axis=1) / jnp.maximum(
            ln, 1.0)

    return pl.pallas_call(
        kern,
        grid=(M // BM,),
        in_specs=[
            pl.BlockSpec((BM, L, D), lambda i: (i, 0, 0)),
            pl.BlockSpec((BM, 1), lambda i: (i, 0)),
        ],
        out_specs=pl.BlockSpec((BM, D), lambda i: (i, 0)),
        out_shape=jax.ShapeDtypeStruct((M, D), jnp.float32),
    )(emb, lens)


def _bpool(node, wg, bg, batch, masks):
    def kern(n_ref, wg_ref, bg_ref, b_ref, m_ref, o_ref):
        node_v = n_ref[...]
        gate = jnp.sum(node_v * wg_ref[...], axis=1, keepdims=True) + bg_ref[0, 0]
        oh = (b_ref[...] == lax.broadcasted_iota(jnp.int32, (N, B), 1))
        ohf = oh.astype(jnp.float32)
        for i in range(NEG + 2):
            mask_c = m_ref[:, i][:, None]
            lg = jnp.where(mask_c > 0, gate, -1e9)
            mseg = jnp.max(jnp.where(oh, lg, -3.4e38), axis=0, keepdims=True)
            mseg = jnp.where(mseg > -1e38, mseg, 0.0)
            mn = jnp.sum(ohf * mseg, axis=1, keepdims=True)
            ex = jnp.exp(lg - mn)
            sb = jax.lax.dot_general(ex, ohf, (((0,), (0,)), ((), ())),
                                     preferred_element_type=jnp.float32)
            sn = jnp.sum(ohf * sb, axis=1, keepdims=True)
            al = ex / (sn + 1e-16)
            w8 = ohf * al
            col = jax.lax.dot_general(w8, node_v, (((0,), (0,)), ((), ())),
                                      preferred_element_type=jnp.float32)
            o_ref[:, i, :] = col

    return pl.pallas_call(
        kern, out_shape=jax.ShapeDtypeStruct((B, NEG + 2, D), jnp.float32))(
            node, wg, bg, batch, masks)


def _head(uth3, Wsa, bsa, vsa, pooled, tth, wa, wb, bwt):
    def kern(u_ref, wsa_ref, bsa_ref, vsa_ref, p_ref, t_ref, wa_ref, wb_ref,
             bwt_ref, o_ref):
        u = u_ref[...]
        t1 = jnp.tanh(
            jnp.dot(u, wsa_ref[...], preferred_element_type=jnp.float32)
            + bsa_ref[...])
        sc = jnp.sum(t1 * vsa_ref[...], axis=1).reshape(B, H)
        m = jnp.max(sc, axis=1, keepdims=True)
        ex = jnp.exp(sc - m)
        aw = ex / jnp.sum(ex, axis=1, keepdims=True)
        u3 = u.reshape(B, H, D)
        u_title = jnp.sum(u3 * aw[:, :, None], axis=1)
        u_graph = p_ref[:, 0, :]
        t_graph = p_ref[:, 1:, :].reshape(B * (NEG + 1), D)
        u_g5 = jnp.broadcast_to(u_graph[:, None, :],
                                (B, NEG + 1, D)).reshape(B * (NEG + 1), D)
        u_t5 = jnp.broadcast_to(u_title[:, None, :],
                                (B, NEG + 1, D)).reshape(B * (NEG + 1), D)
        bwt = bwt_ref[0, 0]
        uw = jax.nn.sigmoid(
            jnp.sum(u_g5 * wa_ref[...], axis=1, keepdims=True)
            + jnp.sum(u_t5 * wb_ref[...], axis=1, keepdims=True) + bwt)
        u_hid = uw * u_g5 + (1.0 - uw) * u_t5
        t_f = t_ref[...]
        tw = jax.nn.sigmoid(
            jnp.sum(t_graph * wa_ref[...], axis=1, keepdims=True)
            + jnp.sum(t_f * wb_ref[...], axis=1, keepdims=True) + bwt)
        t_hid = tw * t_graph + (1.0 - tw) * t_f
        o_ref[...] = jnp.sum(u_hid * t_hid, axis=1).reshape(B, NEG + 1)

    return pl.pallas_call(
        kern, out_shape=jax.ShapeDtypeStruct((B, NEG + 1), jnp.float32))(
            uth3, Wsa, bsa, vsa, pooled, tth, wa, wb, bwt)


def _a8(u, v):
    return jnp.concatenate(
        [u[:, None], v[:, None], jnp.zeros((D, 6), jnp.float32)], axis=1)


def _cross_pair(xs, xd, Ws, Wd, cs, cd, src, dst, Nd, E, K):
    hs, avs = _lin2(xs, Ws, _a8(cs, cs))
    hd, avd = _lin2(xd, Wd, _a8(cd, cd))
    P, s2 = _sc_edge(hs, avs[:, 0], avd[:, 0], src, dst, Nd, E, K)
    return _combine(P, s2)


def kernel(params, x, edge_index, bi_edge_index, y, batch, hist_mask,
           pos_mask, neg_masks, hist_seqs, hist_seq_lens, pos_seq,
           pos_seq_len, neg_seqs, neg_seq_lens, title_table):
    p = params
    src = edge_index[0].astype(jnp.int32)
    dst = edge_index[1].astype(jnp.int32)
    bi_src = bi_edge_index[0].astype(jnp.int32)
    bi_dst = bi_edge_index[1].astype(jnp.int32)

    all_seqs = jnp.concatenate(
        [hist_seqs.reshape(-1), pos_seq[:, 0], neg_seqs.reshape(-1)])
    all_lens = jnp.concatenate(
        [hist_seq_lens.reshape(-1), pos_seq_len, neg_seq_lens.reshape(-1)])
    tt_pad = jnp.pad(title_table.astype(jnp.int32), ((0, 0), (0, 12)))
    ids3 = _sc_gather(tt_pad, all_seqs)[:, :L]
    emb = _sc_gather(p['wt'], ids3.reshape(-1))
    pooled_t = _wpool(emb.reshape(NEWS_N, L, D),
                      all_lens.astype(jnp.float32).reshape(NEWS_N, 1))
    titles = _lin(pooled_t, p['Wt'], p['bt'], True)
    uth = _lin(titles[:B * H], p['Wu1'], p['bu1'], True)
    tth = jnp.concatenate(
        [titles[B * H:B * H + B].reshape(B, 1, D),
         titles[B * H + B:].reshape(B, NEG, D)], axis=1)
    news_h = jnp.concatenate(
        [uth.reshape(B, H, D), tth], axis=1).reshape(NEWS_N, D)

    node0 = _sc_gather(p['ent'], x[:, 0].astype(jnp.int32))

    h1, av1 = _lin2(node0, p['Wg1'], _a8(p['as1'], p['ad1']))
    P1, s1 = _sc_edge(h1, av1[:, 0], av1[:, 1], src, dst, N, EG, 80)
    node_h = _combine(P1, s1)

    node_c = _cross_pair(news_h, node_h, p['Ws1'], p['Wd1'], p['cs1'],
                         p['cd1'], bi_src, bi_dst, N, EBI, 64)
    news_c = _cross_pair(node_h, news_h, p['Ws1'], p['Wd1'], p['cs1'],
                         p['cd1'], bi_dst, bi_src, NEWS_N, EBI, 64)

    nc3 = news_c.reshape(B, H + NEG + 1, D)
    uth2 = _lin(nc3[:, :H].reshape(B * H, D), p['Wu2'], p['bu2'], True)
    news_h2 = jnp.concatenate(
        [uth2.reshape(B, H, D), nc3[:, H:]], axis=1).reshape(NEWS_N, D)

    h2, av2 = _lin2(node_c, p['Wg2'], _a8(p['as2'], p['ad2']))
    P2, s2 = _sc_edge(h2, av2[:, 0], av2[:, 1], src, dst, N, EG, 80)
    node_h2 = _combine(P2, s2)

    node_c2 = _cross_pair(news_h2, node_h2, p['Ws2'], p['Wd2'], p['cs2'],
                          p['cd2'], bi_src, bi_dst, N, EBI, 64)
    news_c2 = _cross_pair(node_h2, news_h2, p['Ws2'], p['Wd2'], p['cs2'],
                          p['cd2'], bi_dst, bi_src, NEWS_N, EBI, 64)

    nc23 = news_c2.reshape(B, H + NEG + 1, D)
    uth3 = _lin(nc23[:, :H].reshape(B * H, D), p['Wu3'], p['bu3'], True)
    tth3 = nc23[:, H:].reshape(B * (NEG + 1), D)

    masks6 = jnp.concatenate(
        [hist_mask, pos_mask, neg_masks], axis=1).astype(jnp.int32)
    pooled = _bpool(node_c2, p['Wgate'][:, 0].reshape(1, D),
                    p['bgate'].reshape(1, 1),
                    batch.astype(jnp.int32).reshape(N, 1), masks6)
    logits = _head(uth3, p['Wsa'], p['bsa'].reshape(1, D),
                   p['vsa'].reshape(1, D), pooled, tth3,
                   p['Wwt'][:D, 0].reshape(1, D), p['Wwt'][D:, 0].reshape(1, D),
                   p['bwt'].reshape(1, 1))
    return logits
